# baseline (speedup 1.0000x reference)
import functools
import math

import jax
import jax.numpy as jnp
from jax import lax
from jax.experimental import pallas as pl
from jax.experimental.pallas import tpu as pltpu
from jax.experimental.pallas import tpu_sc as plsc

N_TOK = 8192
D_MODEL = 768
D_FF = 3072
E = 8
TB = 512
NB = N_TOK // TB
S = NB + E - 1
CHUNK = 128
NCHUNK = N_TOK // CHUNK
SC_W = 64
SC_UNITS = 32

_HIGH = lax.Precision.HIGHEST


def _router_body(x_ref, rw_ref, rb_ref, logits_ref, probs_ref, oh_ref):
    xb = x_ref[...]
    rw = rw_ref[...]
    lt = lax.dot_general(rw, xb, (((1,), (1,)), ((), ())),
                         preferred_element_type=jnp.float32)
    lt = lt + rb_ref[...]
    m = jnp.max(lt, axis=0, keepdims=True)
    ex = jnp.exp(lt - m)
    pt = ex / jnp.sum(ex, axis=0, keepdims=True)
    eq = (pt == jnp.max(pt, axis=0, keepdims=True)).astype(jnp.float32)
    t0 = lax.broadcasted_iota(jnp.int32, (E, E), 0)
    t1 = lax.broadcasted_iota(jnp.int32, (E, E), 1)
    tri = (t1 <= t0).astype(jnp.float32)
    cnt = jnp.dot(tri, eq, precision=_HIGH, preferred_element_type=jnp.float32)
    oht = eq * (cnt == 1.0).astype(jnp.float32)
    logits_ref[...] = lt.T
    probs_ref[...] = pt.T
    oh_ref[...] = oht.T


def _router(x, router_w, router_b2d):
    blk = 1024
    grid = (N_TOK // blk,)
    out_shape = [jax.ShapeDtypeStruct((N_TOK, E), jnp.float32)] * 3
    return pl.pallas_call(
        _router_body,
        grid=grid,
        in_specs=[
            pl.BlockSpec((blk, D_MODEL), lambda i: (i, 0)),
            pl.BlockSpec((E, D_MODEL), lambda i: (0, 0)),
            pl.BlockSpec((E, 1), lambda i: (0, 0)),
        ],
        out_specs=[pl.BlockSpec((blk, E), lambda i: (i, 0))] * 3,
        out_shape=out_shape,
    )(x, router_w, router_b2d)


def _positions_body(oh_ref, sp_ref, meta_ref):
    oh = oh_ref[...]
    ce = CHUNK * E

    ci = lax.broadcasted_iota(jnp.int32, (ce, E), 0)
    ei = lax.broadcasted_iota(jnp.int32, (ce, E), 1)
    A = (ci % E == ei).astype(jnp.float32)
    s = jnp.dot(oh, A, precision=_HIGH, preferred_element_type=jnp.float32)

    g0 = lax.broadcasted_iota(jnp.int32, (NCHUNK, NCHUNK), 0)
    g1 = lax.broadcasted_iota(jnp.int32, (NCHUNK, NCHUNK), 1)
    Lg = (g1 < g0).astype(jnp.float32)
    P = jnp.dot(Lg, s, precision=_HIGH, preferred_element_type=jnp.float32)

    r0 = lax.broadcasted_iota(jnp.int32, (ce, ce), 0)
    r1 = lax.broadcasted_iota(jnp.int32, (ce, ce), 1)
    B = ((r0 % E == r1 % E) & (r0 // E < r1 // E)).astype(jnp.float32)
    w = jnp.dot(oh, B, precision=_HIGH, preferred_element_type=jnp.float32)

    totals = jnp.sum(s, axis=0, keepdims=True)
    e0 = lax.broadcasted_iota(jnp.int32, (E, E), 0)
    e1 = lax.broadcasted_iota(jnp.int32, (E, E), 1)
    Rs = (e0 < e1).astype(jnp.float32)
    offs = jnp.dot(totals, Rs, precision=_HIGH,
                   preferred_element_type=jnp.float32)

    c0 = lax.broadcasted_iota(jnp.int32, (E, ce), 0)
    c1 = lax.broadcasted_iota(jnp.int32, (E, ce), 1)
    C = (c1 % E == c0).astype(jnp.float32)
    P2 = jnp.dot(P, C, precision=_HIGH, preferred_element_type=jnp.float32)
    offs2 = jnp.dot(offs, C, precision=_HIGH, preferred_element_type=jnp.float32)

    val = oh * (P2 + w + offs2)
    d0 = lax.broadcasted_iota(jnp.int32, (ce, CHUNK), 0)
    d1 = lax.broadcasted_iota(jnp.int32, (ce, CHUNK), 1)
    Dm = (d0 // E == d1).astype(jnp.float32)
    sp = jnp.dot(val, Dm, precision=_HIGH, preferred_element_type=jnp.float32)
    sp_ref[...] = sp.astype(jnp.int32)

    mrow = lax.broadcasted_iota(jnp.int32, (E, E), 0)
    meta = jnp.where(mrow == 0, jnp.broadcast_to(totals, (E, E)),
                     jnp.where(mrow == 1, jnp.broadcast_to(offs, (E, E)), 0.0))
    meta_ref[...] = meta.astype(jnp.int32)


def _positions(oh2):
    return pl.pallas_call(
        _positions_body,
        out_shape=[jax.ShapeDtypeStruct((NCHUNK, CHUNK), jnp.int32),
                   jax.ShapeDtypeStruct((E, E), jnp.int32)],
    )(oh2)


def _build_schedule(counts, starts):
    ends = starts + counts
    b = jnp.arange(NB, dtype=jnp.int32)
    bs = b * TB
    ov_lo = jnp.maximum(starts[None, :], bs[:, None])
    ov_hi = jnp.minimum(ends[None, :], (bs + TB)[:, None])
    valid = (ov_lo < ov_hi).reshape(-1)
    pos = jnp.cumsum(valid.astype(jnp.int32)) - 1
    dest = jnp.where(valid, pos, S)
    blk_f = jnp.repeat(b, E)
    exp_f = jnp.tile(jnp.arange(E, dtype=jnp.int32), NB)
    lo_f = (ov_lo - bs[:, None]).reshape(-1)
    hi_f = (ov_hi - bs[:, None]).reshape(-1)
    last_exp = jnp.max(jnp.where(counts > 0,
                                 jnp.arange(E, dtype=jnp.int32), -1))
    fill = [jnp.full((S + 1,), NB - 1, jnp.int32),
            jnp.broadcast_to(last_exp, (S + 1,)).astype(jnp.int32),
            jnp.zeros((S + 1,), jnp.int32),
            jnp.zeros((S + 1,), jnp.int32)]
    rows = [f.at[dest].set(v)[:S]
            for f, v in zip(fill, [blk_f, exp_f, lo_f, hi_f])]
    return jnp.stack(rows)


def _ffn_body(sched_ref, xs_ref, w1_ref, b1_ref, w2_ref, b2_ref, o_ref):
    s = pl.program_id(0)
    e = sched_ref[1, s]
    r_lo = sched_ref[2, s]
    r_hi = sched_ref[3, s]
    @pl.when(r_hi > r_lo)
    def _():
        xb = xs_ref[...]
        h = jnp.dot(xb, w1_ref[0], preferred_element_type=jnp.float32)
        h = h + b1_ref[pl.ds(e, 1), :]
        h = 0.5 * h * (1.0 + lax.erf(h * (1.0 / math.sqrt(2.0))))
        y = jnp.dot(h, w2_ref[0], preferred_element_type=jnp.float32)
        y = y + b2_ref[pl.ds(e, 1), :]

        @pl.when((r_lo == 0) & (r_hi == TB))
        def _():
            o_ref[...] = y

        @pl.when((r_lo > 0) | (r_hi < TB))
        def _():
            rows = lax.broadcasted_iota(jnp.int32, (TB, 1), 0)
            m = (rows >= r_lo) & (rows < r_hi)
            o_ref[...] = jnp.where(m, y, o_ref[...])


def _ffn(sched, xs, W1, b1, W2, b2):
    grid_spec = pltpu.PrefetchScalarGridSpec(
        num_scalar_prefetch=1,
        grid=(S,),
        in_specs=[
            pl.BlockSpec((TB, D_MODEL), lambda s, sc: (sc[0, s], 0)),
            pl.BlockSpec((1, D_MODEL, D_FF), lambda s, sc: (sc[1, s], 0, 0)),
            pl.BlockSpec((E, D_FF), lambda s, sc: (0, 0)),
            pl.BlockSpec((1, D_FF, D_MODEL), lambda s, sc: (sc[1, s], 0, 0)),
            pl.BlockSpec((E, D_MODEL), lambda s, sc: (0, 0)),
        ],
        out_specs=pl.BlockSpec((TB, D_MODEL), lambda s, sc: (sc[0, s], 0)),
    )
    return pl.pallas_call(
        _ffn_body,
        grid_spec=grid_spec,
        out_shape=jax.ShapeDtypeStruct((N_TOK, D_MODEL), jnp.float32),
    )(sched, xs, W1, b1, W2, b2)


@functools.cache
def _vector_mesh():
    return plsc.VectorSubcoreMesh(core_axis_name="core",
                                  subcore_axis_name="subcore")


def _sc_scatter_iota(data, idx):
    n, d = data.shape
    steps = n // 128 // SC_UNITS

    @functools.partial(pl.kernel,
                       out_type=jax.ShapeDtypeStruct((n, d), data.dtype),
                       mesh=_vector_mesh(), scratch_types=[])
    def run(x_hbm, i_hbm, o_hbm):
        def body(x_vmem, i_vmem):
            pltpu.sync_copy(x_vmem, o_hbm.at[i_vmem.at[0]])

        pltpu.emit_pipeline(
            body,
            grid=(SC_UNITS, steps),
            in_specs=[
                pl.BlockSpec((128, d), lambda u, j: (u * steps + j, 0)),
                pl.BlockSpec((1, 128), lambda u, j: (0, u * steps + j)),
            ],
            out_specs=[],
            core_axis_name=("core", "subcore"),
            dimension_semantics=(pltpu.PARALLEL, pltpu.ARBITRARY),
        )(x_hbm, i_hbm)

    return run(data, idx)


def _sc_gather_rows(src, idx):
    n, d = src.shape
    steps = n // SC_W // SC_UNITS

    @functools.partial(pl.kernel,
                       out_type=jax.ShapeDtypeStruct((n, d), src.dtype),
                       mesh=_vector_mesh(), scratch_types=[])
    def run(x_hbm, i_hbm, o_hbm):
        def body(i_vmem, o_vmem):
            pltpu.sync_copy(x_hbm.at[i_vmem.at[0, pl.ds(0, SC_W)]], o_vmem)

        pltpu.emit_pipeline(
            body,
            grid=(SC_UNITS, steps),
            in_specs=[
                pl.BlockSpec((1, 128), lambda u, j: (u * steps + j, 0)),
            ],
            out_specs=[
                pl.BlockSpec((SC_W, d), lambda u, j: (u * steps + j, 0)),
            ],
            core_axis_name=("core", "subcore"),
            dimension_semantics=(pltpu.PARALLEL, pltpu.ARBITRARY),
        )(i_hbm, o_hbm)

    return run(src, idx)


def _pad_idx(v):
    v2 = v.reshape(N_TOK // SC_W, SC_W)
    return jnp.concatenate([v2, jnp.zeros_like(v2)], axis=1)


def kernel(x, router_w, router_b, W1, b1, W2, b2):
    logits, probs, oh = _router(x, router_w, router_b.reshape(E, 1))
    sp2, meta = _positions(oh.reshape(NCHUNK, CHUNK * E))
    sched = _build_schedule(meta[0], meta[1])
    iota128 = jnp.broadcast_to(
        jnp.arange(N_TOK, dtype=jnp.int32)[:, None], (N_TOK, 128))
    inv128 = _sc_scatter_iota(iota128, sp2.reshape(1, N_TOK))
    xs = _sc_gather_rows(x, _pad_idx(inv128[:, 0]))
    ys = _ffn(sched, xs, W1, b1, W2, b2)
    out = _sc_gather_rows(ys, _pad_idx(sp2.reshape(N_TOK)))
    return out, probs, logits


# speedup vs baseline: 1.0096x; 1.0096x over previous
import functools
import math

import jax
import jax.numpy as jnp
from jax import lax
from jax.experimental import pallas as pl
from jax.experimental.pallas import tpu as pltpu
from jax.experimental.pallas import tpu_sc as plsc

N_TOK = 8192
D_MODEL = 768
D_FF = 3072
E = 8
TB = 512
NB = N_TOK // TB
S = NB + E - 1
CHUNK = 128
NCHUNK = N_TOK // CHUNK
SC_W = 64
SC_UNITS = 32

_HIGH = lax.Precision.HIGHEST


def _router_body(x_ref, rw_ref, rb_ref, logits_ref, probs_ref, oh_ref):
    xb = x_ref[...]
    rw = rw_ref[...]
    lt = lax.dot_general(rw, xb, (((1,), (1,)), ((), ())),
                         preferred_element_type=jnp.float32)
    lt = lt + rb_ref[...]
    m = jnp.max(lt, axis=0, keepdims=True)
    ex = jnp.exp(lt - m)
    pt = ex / jnp.sum(ex, axis=0, keepdims=True)
    eq = (pt == jnp.max(pt, axis=0, keepdims=True)).astype(jnp.float32)
    t0 = lax.broadcasted_iota(jnp.int32, (E, E), 0)
    t1 = lax.broadcasted_iota(jnp.int32, (E, E), 1)
    tri = (t1 <= t0).astype(jnp.float32)
    cnt = jnp.dot(tri, eq, precision=_HIGH, preferred_element_type=jnp.float32)
    oht = eq * (cnt == 1.0).astype(jnp.float32)
    logits_ref[...] = lt.T
    probs_ref[...] = pt.T
    oh_ref[...] = oht.T


def _router(x, router_w, router_b2d):
    blk = 1024
    grid = (N_TOK // blk,)
    out_shape = [jax.ShapeDtypeStruct((N_TOK, E), jnp.float32)] * 3
    return pl.pallas_call(
        _router_body,
        grid=grid,
        in_specs=[
            pl.BlockSpec((blk, D_MODEL), lambda i: (i, 0)),
            pl.BlockSpec((E, D_MODEL), lambda i: (0, 0)),
            pl.BlockSpec((E, 1), lambda i: (0, 0)),
        ],
        out_specs=[pl.BlockSpec((blk, E), lambda i: (i, 0))] * 3,
        out_shape=out_shape,
    )(x, router_w, router_b2d)


def _positions_body(oh_ref, sp_ref, meta_ref):
    oh = oh_ref[...]
    ce = CHUNK * E

    ci = lax.broadcasted_iota(jnp.int32, (ce, E), 0)
    ei = lax.broadcasted_iota(jnp.int32, (ce, E), 1)
    A = (ci % E == ei).astype(jnp.float32)
    s = jnp.dot(oh, A, precision=_HIGH, preferred_element_type=jnp.float32)

    g0 = lax.broadcasted_iota(jnp.int32, (NCHUNK, NCHUNK), 0)
    g1 = lax.broadcasted_iota(jnp.int32, (NCHUNK, NCHUNK), 1)
    Lg = (g1 < g0).astype(jnp.float32)
    P = jnp.dot(Lg, s, precision=_HIGH, preferred_element_type=jnp.float32)

    r0 = lax.broadcasted_iota(jnp.int32, (ce, ce), 0)
    r1 = lax.broadcasted_iota(jnp.int32, (ce, ce), 1)
    B = ((r0 % E == r1 % E) & (r0 // E < r1 // E)).astype(jnp.float32)
    w = jnp.dot(oh, B, precision=_HIGH, preferred_element_type=jnp.float32)

    totals = jnp.sum(s, axis=0, keepdims=True)
    e0 = lax.broadcasted_iota(jnp.int32, (E, E), 0)
    e1 = lax.broadcasted_iota(jnp.int32, (E, E), 1)
    Rs = (e0 < e1).astype(jnp.float32)
    offs = jnp.dot(totals, Rs, precision=_HIGH,
                   preferred_element_type=jnp.float32)

    c0 = lax.broadcasted_iota(jnp.int32, (E, ce), 0)
    c1 = lax.broadcasted_iota(jnp.int32, (E, ce), 1)
    C = (c1 % E == c0).astype(jnp.float32)
    P2 = jnp.dot(P, C, precision=_HIGH, preferred_element_type=jnp.float32)
    offs2 = jnp.dot(offs, C, precision=_HIGH, preferred_element_type=jnp.float32)

    val = oh * (P2 + w + offs2)
    d0 = lax.broadcasted_iota(jnp.int32, (ce, CHUNK), 0)
    d1 = lax.broadcasted_iota(jnp.int32, (ce, CHUNK), 1)
    Dm = (d0 // E == d1).astype(jnp.float32)
    sp = jnp.dot(val, Dm, precision=_HIGH, preferred_element_type=jnp.float32)
    sp_ref[...] = sp.astype(jnp.int32)

    mrow = lax.broadcasted_iota(jnp.int32, (E, E), 0)
    meta = jnp.where(mrow == 0, jnp.broadcast_to(totals, (E, E)),
                     jnp.where(mrow == 1, jnp.broadcast_to(offs, (E, E)), 0.0))
    meta_ref[...] = meta.astype(jnp.int32)


def _positions(oh2):
    return pl.pallas_call(
        _positions_body,
        out_shape=[jax.ShapeDtypeStruct((NCHUNK, CHUNK), jnp.int32),
                   jax.ShapeDtypeStruct((E, E), jnp.int32)],
    )(oh2)


def _build_schedule(counts, starts):
    ends = starts + counts
    b = jnp.arange(NB, dtype=jnp.int32)
    bs = b * TB
    ov_lo = jnp.maximum(starts[None, :], bs[:, None])
    ov_hi = jnp.minimum(ends[None, :], (bs + TB)[:, None])
    valid = (ov_lo < ov_hi).reshape(-1)
    pos = jnp.cumsum(valid.astype(jnp.int32)) - 1
    dest = jnp.where(valid, pos, S)
    blk_f = jnp.repeat(b, E)
    exp_f = jnp.tile(jnp.arange(E, dtype=jnp.int32), NB)
    lo_f = (ov_lo - bs[:, None]).reshape(-1)
    hi_f = (ov_hi - bs[:, None]).reshape(-1)
    last_exp = jnp.max(jnp.where(counts > 0,
                                 jnp.arange(E, dtype=jnp.int32), -1))
    fill = [jnp.full((S + 1,), NB - 1, jnp.int32),
            jnp.broadcast_to(last_exp, (S + 1,)).astype(jnp.int32),
            jnp.zeros((S + 1,), jnp.int32),
            jnp.zeros((S + 1,), jnp.int32)]
    rows = [f.at[dest].set(v)[:S]
            for f, v in zip(fill, [blk_f, exp_f, lo_f, hi_f])]
    return jnp.stack(rows)


def _ffn_body(sched_ref, xs_ref, w1_ref, b1_ref, w2_ref, b2_ref, o_ref):
    s = pl.program_id(0)
    e = sched_ref[1, s]
    r_lo = sched_ref[2, s]
    r_hi = sched_ref[3, s]
    xb = xs_ref[...]
    h = jnp.dot(xb, w1_ref[0], preferred_element_type=jnp.float32)
    h = h + b1_ref[pl.ds(e, 1), :]
    h = 0.5 * h * (1.0 + lax.erf(h * (1.0 / math.sqrt(2.0))))
    y = jnp.dot(h, w2_ref[0], preferred_element_type=jnp.float32)
    y = y + b2_ref[pl.ds(e, 1), :]
    rows = lax.broadcasted_iota(jnp.int32, (TB, 1), 0)
    m = (rows >= r_lo) & (rows < r_hi)
    o_ref[...] = jnp.where(m, y, o_ref[...])


def _ffn(sched, xs, W1, b1, W2, b2):
    grid_spec = pltpu.PrefetchScalarGridSpec(
        num_scalar_prefetch=1,
        grid=(S,),
        in_specs=[
            pl.BlockSpec((TB, D_MODEL), lambda s, sc: (sc[0, s], 0)),
            pl.BlockSpec((1, D_MODEL, D_FF), lambda s, sc: (sc[1, s], 0, 0)),
            pl.BlockSpec((E, D_FF), lambda s, sc: (0, 0)),
            pl.BlockSpec((1, D_FF, D_MODEL), lambda s, sc: (sc[1, s], 0, 0)),
            pl.BlockSpec((E, D_MODEL), lambda s, sc: (0, 0)),
        ],
        out_specs=pl.BlockSpec((TB, D_MODEL), lambda s, sc: (sc[0, s], 0)),
    )
    return pl.pallas_call(
        _ffn_body,
        grid_spec=grid_spec,
        out_shape=jax.ShapeDtypeStruct((N_TOK, D_MODEL), jnp.float32),
    )(sched, xs, W1, b1, W2, b2)


@functools.cache
def _vector_mesh():
    return plsc.VectorSubcoreMesh(core_axis_name="core",
                                  subcore_axis_name="subcore")


def _sc_scatter_iota(data, idx):
    n, d = data.shape
    steps = n // 128 // SC_UNITS

    @functools.partial(pl.kernel,
                       out_type=jax.ShapeDtypeStruct((n, d), data.dtype),
                       mesh=_vector_mesh(), scratch_types=[])
    def run(x_hbm, i_hbm, o_hbm):
        def body(x_vmem, i_vmem):
            pltpu.sync_copy(x_vmem, o_hbm.at[i_vmem.at[0]])

        pltpu.emit_pipeline(
            body,
            grid=(SC_UNITS, steps),
            in_specs=[
                pl.BlockSpec((128, d), lambda u, j: (u * steps + j, 0)),
                pl.BlockSpec((1, 128), lambda u, j: (0, u * steps + j)),
            ],
            out_specs=[],
            core_axis_name=("core", "subcore"),
            dimension_semantics=(pltpu.PARALLEL, pltpu.ARBITRARY),
        )(x_hbm, i_hbm)

    return run(data, idx)


def _sc_gather_rows(src, idx):
    n, d = src.shape
    steps = n // SC_W // SC_UNITS

    @functools.partial(pl.kernel,
                       out_type=jax.ShapeDtypeStruct((n, d), src.dtype),
                       mesh=_vector_mesh(), scratch_types=[])
    def run(x_hbm, i_hbm, o_hbm):
        def body(i_vmem, o_vmem):
            pltpu.sync_copy(x_hbm.at[i_vmem.at[0, pl.ds(0, SC_W)]], o_vmem)

        pltpu.emit_pipeline(
            body,
            grid=(SC_UNITS, steps),
            in_specs=[
                pl.BlockSpec((1, 128), lambda u, j: (u * steps + j, 0)),
            ],
            out_specs=[
                pl.BlockSpec((SC_W, d), lambda u, j: (u * steps + j, 0)),
            ],
            core_axis_name=("core", "subcore"),
            dimension_semantics=(pltpu.PARALLEL, pltpu.ARBITRARY),
        )(i_hbm, o_hbm)

    return run(src, idx)


def _pad_idx(v):
    v2 = v.reshape(N_TOK // SC_W, SC_W)
    return jnp.concatenate([v2, jnp.zeros_like(v2)], axis=1)


def kernel(x, router_w, router_b, W1, b1, W2, b2):
    logits, probs, oh = _router(x, router_w, router_b.reshape(E, 1))
    sp2, meta = _positions(oh.reshape(NCHUNK, CHUNK * E))
    sched = _build_schedule(meta[0], meta[1])
    iota128 = jnp.broadcast_to(
        jnp.arange(N_TOK, dtype=jnp.int32)[:, None], (N_TOK, 128))
    inv128 = _sc_scatter_iota(iota128, sp2.reshape(1, N_TOK))
    xs = _sc_gather_rows(x, _pad_idx(inv128[:, 0]))
    ys = _ffn(sched, xs, W1, b1, W2, b2)
    out = _sc_gather_rows(ys, _pad_idx(sp2.reshape(N_TOK)))
    return out, probs, logits
